# baseline (speedup 1.0000x reference)
import functools

import jax
import jax.numpy as jnp
from jax import lax
from jax.experimental import pallas as pl
from jax.experimental.pallas import tpu as pltpu
from jax.experimental.pallas import tpu_sc as plsc

_N = 10000
_E = 160000
_D = 256
_H = 128
_NSUB = 16
_EP = _E // _NSUB
_G = 80
_NCHK = _EP // _G
_DEPTH = 4
_PW = 3 * _G
_RPT = 624
_RPT_LAST = _N - 15 * _RPT


_BN = 1000


def _h_body(w_ref, x_ref, oh_ref, w0_ref, w1_ref, w3_ref, out_ref):
    c = pl.program_id(1)
    sl = pl.ds(c * _H, _H)
    wc = (w_ref[0] * w0_ref[:, sl] + w_ref[1] * w1_ref[:, sl]
          + w_ref[3] * w3_ref[:, sl])
    out_ref[...] = (
        jnp.dot(x_ref[...], wc, preferred_element_type=jnp.float32)
        + w_ref[2] * oh_ref[...])


def _dense_h(x, one_hot_h, weights, w0, w1, w3):
    nbi = _N // _BN
    return pl.pallas_call(
        _h_body,
        grid=(nbi, 2),
        in_specs=[
            pl.BlockSpec(memory_space=pltpu.SMEM),
            pl.BlockSpec((_BN, _D), lambda i, c: (i, 0)),
            pl.BlockSpec((_BN, _H), lambda i, c: (i, c)),
            pl.BlockSpec((_D, _D), lambda i, c: (0, 0)),
            pl.BlockSpec((_D, _D), lambda i, c: (0, 0)),
            pl.BlockSpec((_D, _D), lambda i, c: (0, 0)),
        ],
        out_specs=pl.BlockSpec((_BN, _H), lambda i, c: (c * nbi + i, 0)),
        out_shape=jax.ShapeDtypeStruct((2 * _N, _H), jnp.float32),
    )(weights, x, one_hot_h, w0, w1, w3)


def _sc_body(h_hbm, pidx_hbm, out_hbm,
             r0, r1, r2, r3, ib0, ib1, ib2, ib3,
             sc0, sc1, sc2, sc3, dc0, dc1, dc2, dc3, acc_sh,
             g0, g1, g2, g3, ss0, ss1, ss2, ss3, i0, i1, i2, i3):
    c = lax.axis_index("c")
    s = lax.axis_index("s")
    rows = (r0, r1, r2, r3)
    idxb = (ib0, ib1, ib2, ib3)
    srcs = (sc0, sc1, sc2, sc3)
    dsts = (dc0, dc1, dc2, dc3)
    gsem = (g0, g1, g2, g3)
    ssem = (ss0, ss1, ss2, ss3)
    isem = (i0, i1, i2, i3)
    coff = c * _N
    cbase = s * _NCHK

    def _zero_row(i, carry):
        for j in range(_H // 16):
            r0[i, pl.ds(j * 16, 16)] = jnp.zeros((16,), jnp.float32)
        return carry
    lax.fori_loop(0, _G, _zero_row, 0)
    rb = s * _RPT

    @pl.when(s < _NSUB - 1)
    def _():
        for t in range(7):
            pltpu.sync_copy(r0, acc_sh.at[pl.ds(rb + t * _G, _G)])
        pltpu.sync_copy(r0.at[pl.ds(0, _RPT - 7 * _G)],
                        acc_sh.at[pl.ds(rb + 7 * _G, _RPT - 7 * _G)])

    @pl.when(s == _NSUB - 1)
    def _():
        for t in range(8):
            pltpu.sync_copy(r0, acc_sh.at[pl.ds(rb + t * _G, _G)])

    plsc.subcore_barrier()

    def _load_idx(j, b):
        pltpu.async_copy(pidx_hbm.at[pl.ds((cbase + j) * _PW, _PW)],
                         idxb[b], isem[b])

    def _wait_idx(b):
        pltpu.make_async_copy(pidx_hbm.at[pl.ds(0, _PW)], idxb[b],
                              isem[b]).wait()

    def _prep_src(b):
        for i in range(_G // 16):
            sl = pl.ds(i * 16, 16)
            srcs[b][sl] = idxb[b][sl].astype(jnp.int32) + coff

    def _prep_dst(b):
        for i in range(_G // 16):
            sl = pl.ds(i * 16, 16)
            dsts[b][sl] = idxb[b][pl.ds(_G + i * 16, 16)].astype(jnp.int32)

    def _issue_gather(b):
        pltpu.async_copy(h_hbm.at[srcs[b]], rows[b], gsem[b])

    def _wait_gather(b):
        pltpu.make_async_copy(h_hbm.at[pl.ds(0, _G)], rows[b],
                              gsem[b]).wait()

    def _issue_scatter(b):
        pltpu.async_copy(rows[b], acc_sh.at[dsts[b]], ssem[b], add=True)

    def _wait_scatter(b):
        pltpu.make_async_copy(rows[b], acc_sh.at[pl.ds(0, _G)],
                              ssem[b]).wait()

    def _scale(b):
        def _grp(g, cc):
            m16 = idxb[b][pl.ds(2 * _G + g * 16, 16)]
            for l in range(16):
                m = jnp.full((16,), m16[l], jnp.float32)
                e = g * 16 + l
                for j in range(_H // 16):
                    sl = pl.ds(j * 16, 16)
                    rows[b][e, sl] = rows[b][e, sl] * m
            return cc
        lax.fori_loop(0, _G // 16, _grp, 0)

    for j in range(2):
        pltpu.sync_copy(pidx_hbm.at[pl.ds((cbase + j) * _PW, _PW)], idxb[j])
        _prep_src(j)
        _issue_gather(j)
    _load_idx(2, 2)
    _load_idx(3, 3)

    def _iter(j, b):
        b2 = (b + 2) % _DEPTH
        _wait_gather(b)
        _scale(b)
        _prep_dst(b)
        _issue_scatter(b)

        @pl.when(j <= _NCHK - 3)
        def _():
            @pl.when(j >= 2)
            def _():
                _wait_scatter(b2)
            _wait_idx(b2)
            _prep_src(b2)
            _issue_gather(b2)

        @pl.when(j <= _NCHK - 5)
        def _():
            _load_idx(j + 4, b)

    def _group(k, carry):
        for u in range(_DEPTH):
            _iter(_DEPTH * k + u, u)
        return carry
    lax.fori_loop(0, (_NCHK - 1) // _DEPTH, _group, 0)

    _wait_gather(0)
    _scale(0)
    _prep_dst(0)
    _issue_scatter(0)
    for b in range(_DEPTH):
        _wait_scatter(b)

    plsc.subcore_barrier()

    @pl.when(s < _NSUB - 1)
    def _():
        pltpu.sync_copy(acc_sh.at[pl.ds(rb, _RPT)],
                        out_hbm.at[pl.ds(rb, _RPT), pl.ds(c * _H, _H)])

    @pl.when(s == _NSUB - 1)
    def _():
        pltpu.sync_copy(acc_sh.at[pl.ds(rb, _RPT_LAST)],
                        out_hbm.at[pl.ds(rb, _RPT_LAST), pl.ds(c * _H, _H)])


def _sparse_agg(h2, pidx):
    mesh = plsc.VectorSubcoreMesh(core_axis_name="c", subcore_axis_name="s")
    f = functools.partial(
        pl.kernel,
        out_type=jax.ShapeDtypeStruct((_N, _D), jnp.float32),
        mesh=mesh,
        scratch_types=(
            [pltpu.VMEM((_G, _H), jnp.float32) for _ in range(_DEPTH)]
            + [pltpu.VMEM((_PW,), jnp.float32) for _ in range(_DEPTH)]
            + [pltpu.VMEM((_G,), jnp.int32) for _ in range(_DEPTH)]
            + [pltpu.VMEM((_G,), jnp.int32) for _ in range(_DEPTH)]
            + [pltpu.VMEM_SHARED((_N, _H), jnp.float32)]
            + [pltpu.SemaphoreType.DMA for _ in range(3 * _DEPTH)]
        ),
    )(_sc_body)
    return f(h2, pidx)


def kernel(x, one_hot_h, weights, edge_index, mask_values, W0, W1, W3):
    h2 = _dense_h(x, one_hot_h, weights, W0, W1, W3)
    nch = _E // _G
    pidx = jnp.stack([edge_index[0].astype(jnp.float32).reshape(nch, _G),
                      edge_index[1].astype(jnp.float32).reshape(nch, _G),
                      mask_values.reshape(nch, _G)], axis=1).reshape(-1)
    return _sparse_agg(h2, pidx)


# speedup vs baseline: 1.1125x; 1.1125x over previous
import functools

import jax
import jax.numpy as jnp
from jax import lax
from jax.experimental import pallas as pl
from jax.experimental.pallas import tpu as pltpu
from jax.experimental.pallas import tpu_sc as plsc

_N = 10000
_E = 160000
_D = 256
_H = 128
_NSUB = 16
_EP = _E // _NSUB
_G = 80
_NCHK = _EP // _G
_DEPTH = 4
_RPT = 624
_RPT_LAST = _N - 15 * _RPT


_BN = 1000


def _h_body(w_ref, x_ref, oh_ref, w0_ref, w1_ref, w3_ref, out_ref):
    wc = (w_ref[0] * w0_ref[...] + w_ref[1] * w1_ref[...]
          + w_ref[3] * w3_ref[...])
    out_ref[...] = (
        jnp.dot(x_ref[...], wc, preferred_element_type=jnp.float32)
        + w_ref[2] * oh_ref[...])


def _dense_h(x, one_hot_h, weights, w0, w1, w3):
    nbi = _N // _BN
    return pl.pallas_call(
        _h_body,
        grid=(2, nbi),
        in_specs=[
            pl.BlockSpec(memory_space=pltpu.SMEM),
            pl.BlockSpec((_BN, _D), lambda c, i: (i, 0)),
            pl.BlockSpec((_BN, _H), lambda c, i: (i, c)),
            pl.BlockSpec((_D, _H), lambda c, i: (0, c)),
            pl.BlockSpec((_D, _H), lambda c, i: (0, c)),
            pl.BlockSpec((_D, _H), lambda c, i: (0, c)),
        ],
        out_specs=pl.BlockSpec((_BN, _H), lambda c, i: (c * nbi + i, 0)),
        out_shape=jax.ShapeDtypeStruct((2 * _N, _H), jnp.float32),
    )(weights, x, one_hot_h, w0, w1, w3)


def _sc_body(h_hbm, ei_hbm, mask_hbm, out_hbm,
             r0, r1, r2, r3, sb0, sb1, sb2, sb3, db0, db1, db2, db3,
             mb0, mb1, mb2, mb3, acc_sh,
             g0, g1, g2, g3, ss0, ss1, ss2, ss3,
             i0, i1, i2, i3, dd0, dd1, dd2, dd3):
    c = lax.axis_index("c")
    s = lax.axis_index("s")
    rows = (r0, r1, r2, r3)
    srcb = (sb0, sb1, sb2, sb3)
    dstb = (db0, db1, db2, db3)
    maskb = (mb0, mb1, mb2, mb3)
    gsem = (g0, g1, g2, g3)
    ssem = (ss0, ss1, ss2, ss3)
    isem = (i0, i1, i2, i3)
    dsem = (dd0, dd1, dd2, dd3)
    coff = c * _N
    ebase = s * _EP

    def _zero_row(i, carry):
        for j in range(_H // 16):
            r0[i, pl.ds(j * 16, 16)] = jnp.zeros((16,), jnp.float32)
        return carry
    lax.fori_loop(0, _G, _zero_row, 0)
    rb = s * _RPT

    @pl.when(s < _NSUB - 1)
    def _():
        for t in range(7):
            pltpu.sync_copy(r0, acc_sh.at[pl.ds(rb + t * _G, _G)])
        pltpu.sync_copy(r0.at[pl.ds(0, _RPT - 7 * _G)],
                        acc_sh.at[pl.ds(rb + 7 * _G, _RPT - 7 * _G)])

    @pl.when(s == _NSUB - 1)
    def _():
        for t in range(8):
            pltpu.sync_copy(r0, acc_sh.at[pl.ds(rb + t * _G, _G)])

    plsc.subcore_barrier()

    def _load_sm(j, b):
        e0 = ebase + j * _G
        pltpu.async_copy(ei_hbm.at[pl.ds(e0, _G)], srcb[b], isem[b])
        pltpu.async_copy(mask_hbm.at[pl.ds(e0, _G)], maskb[b], isem[b])

    def _wait_sm(b):
        pltpu.make_async_copy(ei_hbm.at[pl.ds(0, _G)], srcb[b],
                              isem[b]).wait()
        pltpu.make_async_copy(mask_hbm.at[pl.ds(0, _G)], maskb[b],
                              isem[b]).wait()

    def _load_dst(j, b):
        e0 = _E + ebase + j * _G
        pltpu.async_copy(ei_hbm.at[pl.ds(e0, _G)], dstb[b], dsem[b])

    def _wait_dst(b):
        pltpu.make_async_copy(ei_hbm.at[pl.ds(0, _G)], dstb[b],
                              dsem[b]).wait()

    def _issue_gather(b):
        for i in range(_G // 16):
            sl = pl.ds(i * 16, 16)
            srcb[b][sl] = srcb[b][sl] + coff
        pltpu.async_copy(h_hbm.at[srcb[b]], rows[b], gsem[b])

    def _wait_gather(b):
        pltpu.make_async_copy(h_hbm.at[pl.ds(0, _G)], rows[b],
                              gsem[b]).wait()

    def _issue_scatter(b):
        pltpu.async_copy(rows[b], acc_sh.at[dstb[b]], ssem[b], add=True)

    def _wait_scatter(b):
        pltpu.make_async_copy(rows[b], acc_sh.at[pl.ds(0, _G)],
                              ssem[b]).wait()

    def _scale(b):
        def _grp(g, cc):
            m16 = maskb[b][pl.ds(g * 16, 16)]
            for l in range(16):
                m = jnp.full((16,), m16[l], jnp.float32)
                e = g * 16 + l
                for j in range(_H // 16):
                    sl = pl.ds(j * 16, 16)
                    rows[b][e, sl] = rows[b][e, sl] * m
            return cc
        lax.fori_loop(0, _G // 16, _grp, 0)

    for j in range(2):
        e0 = ebase + j * _G
        pltpu.sync_copy(ei_hbm.at[pl.ds(e0, _G)], srcb[j])
        pltpu.sync_copy(mask_hbm.at[pl.ds(e0, _G)], maskb[j])
        pltpu.sync_copy(ei_hbm.at[pl.ds(_E + e0, _G)], dstb[j])
        _issue_gather(j)
    _load_sm(2, 2)
    _load_sm(3, 3)

    def _iter(j, b):
        b2 = (b + 2) % _DEPTH
        _wait_gather(b)
        _scale(b)

        @pl.when(j >= 2)
        def _():
            _wait_dst(b)
        _issue_scatter(b)

        @pl.when(j <= _NCHK - 3)
        def _():
            @pl.when(j >= 2)
            def _():
                _wait_scatter(b2)
            _load_dst(j + 2, b2)
            _wait_sm(b2)
            _issue_gather(b2)

        @pl.when(j <= _NCHK - 5)
        def _():
            _load_sm(j + 4, b)

    def _group(k, carry):
        for u in range(_DEPTH):
            _iter(_DEPTH * k + u, u)
        return carry
    lax.fori_loop(0, (_NCHK - 1) // _DEPTH, _group, 0)

    _wait_gather(0)
    _scale(0)
    _wait_dst(0)
    _issue_scatter(0)
    for b in range(_DEPTH):
        _wait_scatter(b)

    plsc.subcore_barrier()

    @pl.when(s < _NSUB - 1)
    def _():
        pltpu.sync_copy(acc_sh.at[pl.ds(rb, _RPT)],
                        out_hbm.at[pl.ds(rb, _RPT), pl.ds(c * _H, _H)])

    @pl.when(s == _NSUB - 1)
    def _():
        pltpu.sync_copy(acc_sh.at[pl.ds(rb, _RPT_LAST)],
                        out_hbm.at[pl.ds(rb, _RPT_LAST), pl.ds(c * _H, _H)])


def _sparse_agg(h2, edge_index, mask_values):
    mesh = plsc.VectorSubcoreMesh(core_axis_name="c", subcore_axis_name="s")
    f = functools.partial(
        pl.kernel,
        out_type=jax.ShapeDtypeStruct((_N, _D), jnp.float32),
        mesh=mesh,
        scratch_types=(
            [pltpu.VMEM((_G, _H), jnp.float32) for _ in range(_DEPTH)]
            + [pltpu.VMEM((_G,), jnp.int32) for _ in range(_DEPTH)]
            + [pltpu.VMEM((_G,), jnp.int32) for _ in range(_DEPTH)]
            + [pltpu.VMEM((_G,), jnp.float32) for _ in range(_DEPTH)]
            + [pltpu.VMEM_SHARED((_N, _H), jnp.float32)]
            + [pltpu.SemaphoreType.DMA for _ in range(4 * _DEPTH)]
        ),
    )(_sc_body)
    return f(h2, edge_index.reshape(-1), mask_values)


def kernel(x, one_hot_h, weights, edge_index, mask_values, W0, W1, W3):
    h2 = _dense_h(x, one_hot_h, weights, W0, W1, W3)
    return _sparse_agg(h2, edge_index, mask_values)
